# Initial kernel scaffold; baseline (speedup 1.0000x reference)
import jax
import jax.numpy as jnp
from jax.experimental import pallas as pl


def kernel(x, encoder_weights, encoder_bias, decoder_bias):
    raise NotImplementedError("write your pallas kernel here")


# speedup vs baseline: 19.0912x; 19.0912x over previous
import functools

import jax
import jax.numpy as jnp
from jax.experimental import pallas as pl

K = 15


def _fused_body(x_ref, w_ref, wt_ref, eb_ref, db_ref, rec_ref, zs_ref):
    x = x_ref[...]
    w = w_ref[...]
    z = jnp.dot(x, w, preferred_element_type=jnp.float32) + eb_ref[...][None, :]
    a = jnp.abs(z)
    m = jnp.max(a, axis=1, keepdims=True)
    for _ in range(K - 1):
        m = jnp.max(jnp.where(a < m, a, -1.0), axis=1, keepdims=True)
    zs = jnp.where(a >= m, z, 0.0)
    zs_ref[...] = zs
    rec_ref[...] = (
        jnp.dot(zs, wt_ref[...], preferred_element_type=jnp.float32)
        + db_ref[...][None, :]
    )


@jax.jit
def kernel(x, encoder_weights, encoder_bias, decoder_bias):
    batch, input_dim = x.shape
    latent_dim = encoder_weights.shape[1]
    blk = min(256, batch)
    grid = (batch // blk,)
    wt = encoder_weights.T
    rec, zs = pl.pallas_call(
        _fused_body,
        grid=grid,
        in_specs=[
            pl.BlockSpec((blk, input_dim), lambda i: (i, 0)),
            pl.BlockSpec((input_dim, latent_dim), lambda i: (0, 0)),
            pl.BlockSpec((latent_dim, input_dim), lambda i: (0, 0)),
            pl.BlockSpec((latent_dim,), lambda i: (0,)),
            pl.BlockSpec((input_dim,), lambda i: (0,)),
        ],
        out_specs=[
            pl.BlockSpec((blk, input_dim), lambda i: (i, 0)),
            pl.BlockSpec((blk, latent_dim), lambda i: (i, 0)),
        ],
        out_shape=[
            jax.ShapeDtypeStruct((batch, input_dim), jnp.float32),
            jax.ShapeDtypeStruct((batch, latent_dim), jnp.float32),
        ],
    )(x, encoder_weights, wt, encoder_bias, decoder_bias)
    return rec, zs


# speedup vs baseline: 19.5266x; 1.0228x over previous
import functools

import jax
import jax.numpy as jnp
from jax.experimental import pallas as pl

K = 15


def _fused_body(x_ref, w_ref, wt_ref, eb_ref, db_ref, rec_ref, zs_ref):
    x = x_ref[...]
    w = w_ref[...]
    z = jnp.dot(x, w, preferred_element_type=jnp.float32) + eb_ref[...][None, :]
    a = jnp.abs(z)
    q = a.shape[1] // 4
    ch = [a[:, i * q : (i + 1) * q] for i in range(4)]

    def ce(x, y):
        return jnp.maximum(x, y), jnp.minimum(x, y)

    A, B = ce(ch[0], ch[1])
    C, D = ce(ch[2], ch[3])
    A, C = ce(A, C)
    B, D = ce(B, D)
    B, C = ce(B, C)
    m = jnp.max(A, axis=1, keepdims=True)
    for _ in range(K - 1):
        sel = A == m
        A = jnp.where(sel, B, A)
        B = jnp.where(sel, C, B)
        C = jnp.where(sel, D, C)
        D = jnp.where(sel, -1.0, D)
        m = jnp.max(A, axis=1, keepdims=True)
    zs = jnp.where(a >= m, z, 0.0)
    zs_ref[...] = zs
    rec_ref[...] = (
        jnp.dot(zs, wt_ref[...], preferred_element_type=jnp.float32)
        + db_ref[...][None, :]
    )


@jax.jit
def kernel(x, encoder_weights, encoder_bias, decoder_bias):
    batch, input_dim = x.shape
    latent_dim = encoder_weights.shape[1]
    blk = min(256, batch)
    grid = (batch // blk,)
    wt = encoder_weights.T
    rec, zs = pl.pallas_call(
        _fused_body,
        grid=grid,
        in_specs=[
            pl.BlockSpec((blk, input_dim), lambda i: (i, 0)),
            pl.BlockSpec((input_dim, latent_dim), lambda i: (0, 0)),
            pl.BlockSpec((latent_dim, input_dim), lambda i: (0, 0)),
            pl.BlockSpec((latent_dim,), lambda i: (0,)),
            pl.BlockSpec((input_dim,), lambda i: (0,)),
        ],
        out_specs=[
            pl.BlockSpec((blk, input_dim), lambda i: (i, 0)),
            pl.BlockSpec((blk, latent_dim), lambda i: (i, 0)),
        ],
        out_shape=[
            jax.ShapeDtypeStruct((batch, input_dim), jnp.float32),
            jax.ShapeDtypeStruct((batch, latent_dim), jnp.float32),
        ],
    )(x, encoder_weights, wt, encoder_bias, decoder_bias)
    return rec, zs


# speedup vs baseline: 26.5029x; 1.3573x over previous
import jax
import jax.numpy as jnp
from jax.experimental import pallas as pl

K = 15
NSLOT = 32


def _ce(s, i, j):
    hi = jnp.maximum(s[i], s[j])
    lo = jnp.minimum(s[i], s[j])
    s[i] = hi
    s[j] = lo


def _bitonic_sort_desc(s, lo, n):
    k = 2
    while k <= n:
        j = k // 2
        while j >= 1:
            for ri in range(n):
                rl = ri ^ j
                if rl > ri:
                    if (ri & k) == 0:
                        _ce(s, lo + ri, lo + rl)
                    else:
                        _ce(s, lo + rl, lo + ri)
            j //= 2
        k *= 2


def _top16_sorted(s):
    _bitonic_sort_desc(s, 0, 16)
    _bitonic_sort_desc(s, 16, 16)
    c = [jnp.maximum(s[i], s[31 - i]) for i in range(16)]
    for j in (8, 4, 2, 1):
        for i in range(16):
            l = i ^ j
            if l > i:
                _ce(c, i, l)
    return c


def _fused_body(x_ref, w_ref, wt_ref, eb_ref, db_ref, rec_ref, zs_ref):
    z = (
        jnp.dot(x_ref[...], w_ref[...], preferred_element_type=jnp.float32)
        + eb_ref[...][None, :]
    )
    a = jnp.abs(z)
    s = [a[:, i * 128 : (i + 1) * 128] for i in range(NSLOT)]
    s = _top16_sorted(s)
    m = jnp.max(s[0], axis=1, keepdims=True)
    for k in range(1, K):
        sel = s[0] == m
        for t in range(K - k):
            s[t] = jnp.where(sel, s[t + 1], s[t])
        m = jnp.max(s[0], axis=1, keepdims=True)
    zs = jnp.where(a >= m, z, 0.0)
    zs_ref[...] = zs
    rec_ref[...] = (
        jnp.dot(
            zs.astype(jnp.bfloat16), wt_ref[...], preferred_element_type=jnp.float32
        )
        + db_ref[...][None, :]
    )


@jax.jit
def kernel(x, encoder_weights, encoder_bias, decoder_bias):
    batch, input_dim = x.shape
    latent_dim = encoder_weights.shape[1]
    blk = min(256, batch)
    grid = (batch // blk,)
    wb = encoder_weights.astype(jnp.bfloat16)
    wtb = wb.T
    xb = x.astype(jnp.bfloat16)
    rec, zs = pl.pallas_call(
        _fused_body,
        grid=grid,
        in_specs=[
            pl.BlockSpec((blk, input_dim), lambda i: (i, 0)),
            pl.BlockSpec((input_dim, latent_dim), lambda i: (0, 0)),
            pl.BlockSpec((latent_dim, input_dim), lambda i: (0, 0)),
            pl.BlockSpec((latent_dim,), lambda i: (0,)),
            pl.BlockSpec((input_dim,), lambda i: (0,)),
        ],
        out_specs=[
            pl.BlockSpec((blk, input_dim), lambda i: (i, 0)),
            pl.BlockSpec((blk, latent_dim), lambda i: (i, 0)),
        ],
        out_shape=[
            jax.ShapeDtypeStruct((batch, input_dim), jnp.float32),
            jax.ShapeDtypeStruct((batch, latent_dim), jnp.float32),
        ],
    )(xb, wb, wtb, encoder_bias, decoder_bias)
    return rec, zs


# speedup vs baseline: 26.6268x; 1.0047x over previous
import jax
import jax.numpy as jnp
from jax.experimental import pallas as pl
from jax.experimental.pallas import tpu as pltpu

K = 15
NSLOT = 32


def _ce(s, i, j):
    hi = jnp.maximum(s[i], s[j])
    lo = jnp.minimum(s[i], s[j])
    s[i] = hi
    s[j] = lo


def _bitonic_sort_desc(s, lo, n):
    k = 2
    while k <= n:
        j = k // 2
        while j >= 1:
            for ri in range(n):
                rl = ri ^ j
                if rl > ri:
                    if (ri & k) == 0:
                        _ce(s, lo + ri, lo + rl)
                    else:
                        _ce(s, lo + rl, lo + ri)
            j //= 2
        k *= 2


def _top16_sorted(s):
    _bitonic_sort_desc(s, 0, 16)
    _bitonic_sort_desc(s, 16, 16)
    c = [jnp.maximum(s[i], s[31 - i]) for i in range(16)]
    for j in (8, 4, 2, 1):
        for i in range(16):
            l = i ^ j
            if l > i:
                _ce(c, i, l)
    return c


def _threshold(a_slots):
    s = _top16_sorted(a_slots)
    m = jnp.max(s[0], axis=1, keepdims=True)
    for k in range(1, K):
        sel = s[0] == m
        for t in range(K - k):
            s[t] = jnp.where(sel, s[t + 1], s[t])
        m = jnp.max(s[0], axis=1, keepdims=True)
    return m


def _pipelined_body(
    x_ref, w_ref, wt_ref, eb_ref, db_ref, rec_ref, zs_ref, zbuf_ref
):
    i = pl.program_id(0)
    n = pl.num_programs(0)

    @pl.when(i < n - 1)
    def _encode():
        zbuf_ref[i % 2] = (
            jnp.dot(x_ref[...], w_ref[...], preferred_element_type=jnp.float32)
            + eb_ref[...][None, :]
        )

    @pl.when(i > 0)
    def _process():
        zv = zbuf_ref[(i + 1) % 2]
        a = jnp.abs(zv)
        s = [a[:, t * 128 : (t + 1) * 128] for t in range(NSLOT)]
        m = _threshold(s)
        zs = jnp.where(a >= m, zv, 0.0)
        zs_ref[...] = zs
        rec_ref[...] = (
            jnp.dot(
                zs.astype(jnp.bfloat16),
                wt_ref[...],
                preferred_element_type=jnp.float32,
            )
            + db_ref[...][None, :]
        )


@jax.jit
def kernel(x, encoder_weights, encoder_bias, decoder_bias):
    batch, input_dim = x.shape
    latent_dim = encoder_weights.shape[1]
    blk = min(256, batch)
    nblk = batch // blk
    grid = (nblk + 1,)
    wb = encoder_weights.astype(jnp.bfloat16)
    wtb = wb.T
    xb = x.astype(jnp.bfloat16)
    rec, zs = pl.pallas_call(
        _pipelined_body,
        grid=grid,
        in_specs=[
            pl.BlockSpec((blk, input_dim), lambda i: (jnp.minimum(i, nblk - 1), 0)),
            pl.BlockSpec((input_dim, latent_dim), lambda i: (0, 0)),
            pl.BlockSpec((latent_dim, input_dim), lambda i: (0, 0)),
            pl.BlockSpec((latent_dim,), lambda i: (0,)),
            pl.BlockSpec((input_dim,), lambda i: (0,)),
        ],
        out_specs=[
            pl.BlockSpec((blk, input_dim), lambda i: (jnp.maximum(i, 1) - 1, 0)),
            pl.BlockSpec((blk, latent_dim), lambda i: (jnp.maximum(i, 1) - 1, 0)),
        ],
        out_shape=[
            jax.ShapeDtypeStruct((batch, input_dim), jnp.float32),
            jax.ShapeDtypeStruct((batch, latent_dim), jnp.float32),
        ],
        scratch_shapes=[pltpu.VMEM((2, blk, latent_dim), jnp.float32)],
    )(xb, wb, wtb, encoder_bias, decoder_bias)
    return rec, zs
